# baseline (speedup 1.0000x reference)
import functools
import math

import jax
import jax.numpy as jnp
from jax.experimental import pallas as pl
from jax.experimental.pallas import tpu as pltpu
from jax.experimental.pallas import tpu_sc as plsc

D_MODEL = 1024
N_EXPERTS = 8
TOP_K = 2
D_FF = 4 * D_MODEL

BLK = 256
_SQRT_HALF = 1.0 / math.sqrt(2.0)


def _ffn_body(gid_ref, xs_ref, w1_ref, b1_ref, w2_ref, b2_ref, out_ref):
    n_blocks = gid_ref.shape[0] - 1

    @pl.when(pl.program_id(0) < gid_ref[n_blocks])
    def _():
        xs = xs_ref[...].astype(w1_ref.dtype)
        h = jnp.dot(xs, w1_ref[0], preferred_element_type=jnp.float32)
        h = h + b1_ref[0, 0].astype(jnp.float32)
        h = 0.5 * h * (1.0 + jax.lax.erf(h * _SQRT_HALF))
        y = jnp.dot(h.astype(w2_ref.dtype), w2_ref[0],
                    preferred_element_type=jnp.float32)
        out_ref[...] = y + b2_ref[0, 0].astype(jnp.float32)


def _grouped_ffn(gid, xs, W1, b1, W2, b2, n_blocks, interpret=False):
    grid_spec = pltpu.PrefetchScalarGridSpec(
        num_scalar_prefetch=1,
        grid=(n_blocks,),
        in_specs=[
            pl.BlockSpec((BLK, D_MODEL), lambda i, gid: (i, 0)),
            pl.BlockSpec((1, D_MODEL, D_FF), lambda i, gid: (gid[i], 0, 0)),
            pl.BlockSpec((1, 1, D_FF), lambda i, gid: (gid[i], 0, 0)),
            pl.BlockSpec((1, D_FF, D_MODEL), lambda i, gid: (gid[i], 0, 0)),
            pl.BlockSpec((1, 1, D_MODEL), lambda i, gid: (gid[i], 0, 0)),
        ],
        out_specs=pl.BlockSpec((BLK, D_MODEL), lambda i, gid: (i, 0)),
    )
    return pl.pallas_call(
        _ffn_body,
        grid_spec=grid_spec,
        out_shape=jax.ShapeDtypeStruct((n_blocks * BLK, D_MODEL), jnp.float32),
        compiler_params=pltpu.CompilerParams(
            dimension_semantics=("arbitrary",)),
        interpret=interpret,
    )(gid, xs, W1, b1, W2, b2)


def _router_body(x_ref, wr_ref, br_ref, duo_ref, cnt_ref):
    logits = jnp.dot(x_ref[...], wr_ref[...],
                     preferred_element_type=jnp.float32) + br_ref[...]
    m = jnp.max(logits, axis=-1, keepdims=True)
    ex = jnp.exp(logits - m)
    sm = ex / jnp.sum(ex, axis=-1, keepdims=True)

    lane = jax.lax.broadcasted_iota(jnp.int32, sm.shape, 1)
    i1 = jnp.argmax(sm, axis=-1, keepdims=True)
    v1 = jnp.max(sm, axis=-1, keepdims=True)
    oh1 = (lane == i1)
    sm2 = jnp.where(oh1, -1.0, sm)
    i2 = jnp.argmax(sm2, axis=-1, keepdims=True)
    v2 = jnp.max(sm2, axis=-1, keepdims=True)
    oh2 = (lane == i2)
    s = v1 + v2
    w1, w2 = v1 / s, v2 / s

    ohs = oh1.astype(jnp.float32) + oh2.astype(jnp.float32)
    c = ohs
    k = 1
    while k < c.shape[0]:
        c = c + jnp.concatenate(
            [jnp.zeros((k, c.shape[1]), c.dtype), c[:-k]], axis=0)
        k *= 2
    excl = c - ohs
    cnt = jnp.sum(ohs, axis=0, keepdims=True)
    cnt_ref[...] = cnt

    pc = jnp.ceil(cnt * (1.0 / BLK)) * BLK
    cum = pc
    for kk in (1, 2, 4):
        cum = cum + jnp.concatenate(
            [jnp.zeros((1, kk), cum.dtype), cum[:, :-kk]], axis=1)
    pad_off = cum - pc

    rank1 = jnp.sum(excl * oh1, axis=-1, keepdims=True)
    rank2 = jnp.sum(excl * oh2, axis=-1, keepdims=True)
    off1 = jnp.sum(pad_off * oh1, axis=-1, keepdims=True)
    off2 = jnp.sum(pad_off * oh2, axis=-1, keepdims=True)
    d1 = off1 + rank1
    d2 = off2 + rank2

    duo_ref[...] = (jnp.where(lane == 0, d1, 0.0)
                    + jnp.where(lane == 1, d2, 0.0)
                    + jnp.where(lane == 2, w1, 0.0)
                    + jnp.where(lane == 3, w2, 0.0))


def _router(xf, Wr, br, interpret=False):
    N = xf.shape[0]
    wr_pad = jnp.zeros((D_MODEL, 128), Wr.dtype).at[:, :N_EXPERTS].set(Wr)
    br_pad = jnp.full((1, 128), -1e30, br.dtype).at[0, :N_EXPERTS].set(br)
    duo, cnt = pl.pallas_call(
        _router_body,
        out_shape=(jax.ShapeDtypeStruct((N, 128), jnp.float32),
                   jax.ShapeDtypeStruct((1, 128), jnp.float32)),
        interpret=interpret,
    )(xf, wr_pad, br_pad)
    return duo, cnt


_SC_WIN = 128
_SC_SUB = 256


def _sc_dispatch(xw, tok, dest, R):
    A2 = dest.shape[1]
    W = xw.shape[1]
    xf2 = xw.reshape(-1, _SC_SUB)
    mesh = plsc.VectorSubcoreMesh(core_axis_name="core",
                                  subcore_axis_name="subcore")

    @functools.partial(
        pl.kernel,
        out_type=jax.ShapeDtypeStruct((R * (W // _SC_SUB), _SC_SUB),
                                      xw.dtype),
        mesh=mesh,
        scratch_types=[pltpu.VMEM((_SC_WIN, _SC_SUB), xw.dtype)])
    def k(x_hbm, tok_hbm, dest_hbm, xs_hbm, buf):
        def body(tok_vmem, dest_vmem):
            pltpu.sync_copy(x_hbm.at[tok_vmem.at[0]], buf)
            pltpu.sync_copy(buf, xs_hbm.at[dest_vmem.at[0]])

        pltpu.emit_pipeline(
            body,
            grid=(A2 // _SC_WIN,),
            in_specs=[pl.BlockSpec((1, _SC_WIN), lambda i: (0, i)),
                      pl.BlockSpec((1, _SC_WIN), lambda i: (0, i))],
            out_specs=[],
            core_axis_name=('core', 'subcore'),
            dimension_semantics=(pltpu.PARALLEL,),
        )(tok_hbm, dest_hbm)

    return k(xf2, tok, dest).reshape(R, W)


def kernel(x, Wr, br, W1, b1, W2, b2, interpret=False):
    B, L, D = x.shape
    xf = x.reshape(-1, D)
    N = xf.shape[0]
    A = N * TOP_K

    n_blocks = (A + N_EXPERTS * (BLK - 1) + BLK - 1) // BLK
    R = n_blocks * BLK

    duo, cnt = _router(xf, Wr, br, interpret=interpret)
    pos = duo[:, :TOP_K].astype(jnp.int32)
    tkw = duo[:, TOP_K:2 * TOP_K]
    dest = pos.reshape(-1)
    tok = jnp.arange(A, dtype=jnp.int32) // TOP_K

    pc = jnp.ceil(cnt[0, :N_EXPERTS] * (1.0 / BLK)) * BLK
    cum_pc = jnp.cumsum(pc)
    n_real = (cum_pc[-1] * (1.0 / BLK)).astype(jnp.int32)
    gid = jnp.minimum(
        jnp.searchsorted(
            cum_pc, (jnp.arange(n_blocks) * BLK).astype(jnp.float32),
            side='right'),
        N_EXPERTS - 1).astype(jnp.int32)
    gid_last = jnp.take(gid, jnp.maximum(n_real - 1, 0))
    gid = jnp.where(jnp.arange(n_blocks) < n_real, gid, gid_last)
    gid = jnp.concatenate([gid, n_real[None]])

    if interpret:
        xs = jnp.zeros((R, D), x.dtype).at[dest].set(xf[tok])
    else:
        xb = jax.lax.bitcast_convert_type(
            xf.astype(jnp.bfloat16).reshape(N, D // 2, 2), jnp.int32)
        nsub = (D // 2) // _SC_SUB
        sub = jnp.arange(nsub, dtype=jnp.int32)
        tok2 = (tok[:, None] * nsub + sub).reshape(1, A * nsub)
        dest2 = (dest[:, None] * nsub + sub).reshape(1, A * nsub)
        xs_i = _sc_dispatch(xb, tok2, dest2, R)
        xs = jax.lax.bitcast_convert_type(
            xs_i, jnp.bfloat16).reshape(R, D)

    bf = jnp.bfloat16
    ys = _grouped_ffn(gid, xs, W1.astype(bf),
                      b1.reshape(N_EXPERTS, 1, D_FF), W2.astype(bf),
                      b2.reshape(N_EXPERTS, 1, D_MODEL), n_blocks,
                      interpret=interpret)

    out = ys[pos[:, 0]] * tkw[:, :1] + ys[pos[:, 1]] * tkw[:, 1:]
    return out.reshape(B, L, D)


# speedup vs baseline: 1.4043x; 1.4043x over previous
import functools
import math

import jax
import jax.numpy as jnp
from jax.experimental import pallas as pl
from jax.experimental.pallas import tpu as pltpu
from jax.experimental.pallas import tpu_sc as plsc

D_MODEL = 1024
N_EXPERTS = 8
TOP_K = 2
D_FF = 4 * D_MODEL

BLK = 256
_SQRT_HALF = 1.0 / math.sqrt(2.0)


def _ffn_body(gid_ref, xs_ref, w1_ref, b1_ref, w2_ref, b2_ref, out_ref):
    n_blocks = gid_ref.shape[0] - 1

    @pl.when(pl.program_id(0) < gid_ref[n_blocks])
    def _():
        xs = xs_ref[...].astype(w1_ref.dtype)
        h = jnp.dot(xs, w1_ref[0], preferred_element_type=jnp.float32)
        h = h + b1_ref[0, 0].astype(jnp.float32)
        h = 0.5 * h * (1.0 + jax.lax.erf(h * _SQRT_HALF))
        y = jnp.dot(h.astype(w2_ref.dtype), w2_ref[0],
                    preferred_element_type=jnp.float32)
        out_ref[...] = y + b2_ref[0, 0].astype(jnp.float32)


def _grouped_ffn(gid, xs, W1, b1, W2, b2, n_blocks, interpret=False):
    grid_spec = pltpu.PrefetchScalarGridSpec(
        num_scalar_prefetch=1,
        grid=(n_blocks,),
        in_specs=[
            pl.BlockSpec((BLK, D_MODEL), lambda i, gid: (i, 0)),
            pl.BlockSpec((1, D_MODEL, D_FF), lambda i, gid: (gid[i], 0, 0)),
            pl.BlockSpec((1, 1, D_FF), lambda i, gid: (gid[i], 0, 0)),
            pl.BlockSpec((1, D_FF, D_MODEL), lambda i, gid: (gid[i], 0, 0)),
            pl.BlockSpec((1, 1, D_MODEL), lambda i, gid: (gid[i], 0, 0)),
        ],
        out_specs=pl.BlockSpec((BLK, D_MODEL), lambda i, gid: (i, 0)),
    )
    return pl.pallas_call(
        _ffn_body,
        grid_spec=grid_spec,
        out_shape=jax.ShapeDtypeStruct((n_blocks * BLK, D_MODEL), jnp.float32),
        compiler_params=pltpu.CompilerParams(
            dimension_semantics=("arbitrary",)),
        interpret=interpret,
    )(gid, xs, W1, b1, W2, b2)


def _router_body(x_ref, wr_ref, br_ref, duo_ref, cnt_ref):
    logits = jnp.dot(x_ref[...], wr_ref[...],
                     preferred_element_type=jnp.float32) + br_ref[...]
    m = jnp.max(logits, axis=-1, keepdims=True)
    ex = jnp.exp(logits - m)
    sm = ex / jnp.sum(ex, axis=-1, keepdims=True)

    lane = jax.lax.broadcasted_iota(jnp.int32, sm.shape, 1)
    i1 = jnp.argmax(sm, axis=-1, keepdims=True)
    v1 = jnp.max(sm, axis=-1, keepdims=True)
    oh1 = (lane == i1)
    sm2 = jnp.where(oh1, -1.0, sm)
    i2 = jnp.argmax(sm2, axis=-1, keepdims=True)
    v2 = jnp.max(sm2, axis=-1, keepdims=True)
    oh2 = (lane == i2)
    s = v1 + v2
    w1, w2 = v1 / s, v2 / s

    ohs = oh1.astype(jnp.float32) + oh2.astype(jnp.float32)
    c = ohs
    k = 1
    while k < c.shape[0]:
        c = c + jnp.concatenate(
            [jnp.zeros((k, c.shape[1]), c.dtype), c[:-k]], axis=0)
        k *= 2
    excl = c - ohs
    cnt = jnp.sum(ohs, axis=0, keepdims=True)
    cnt_ref[...] = cnt

    pc = jnp.ceil(cnt * (1.0 / BLK)) * BLK
    cum = pc
    for kk in (1, 2, 4):
        cum = cum + jnp.concatenate(
            [jnp.zeros((1, kk), cum.dtype), cum[:, :-kk]], axis=1)
    pad_off = cum - pc

    rank1 = jnp.sum(excl * oh1, axis=-1, keepdims=True)
    rank2 = jnp.sum(excl * oh2, axis=-1, keepdims=True)
    off1 = jnp.sum(pad_off * oh1, axis=-1, keepdims=True)
    off2 = jnp.sum(pad_off * oh2, axis=-1, keepdims=True)
    d1 = off1 + rank1
    d2 = off2 + rank2

    duo_ref[...] = (jnp.where(lane == 0, d1, 0.0)
                    + jnp.where(lane == 1, d2, 0.0)
                    + jnp.where(lane == 2, w1, 0.0)
                    + jnp.where(lane == 3, w2, 0.0))


def _router(xf, Wr, br, interpret=False):
    N = xf.shape[0]
    wr_pad = jnp.zeros((D_MODEL, 128), Wr.dtype).at[:, :N_EXPERTS].set(Wr)
    br_pad = jnp.full((1, 128), -1e30, br.dtype).at[0, :N_EXPERTS].set(br)
    duo, cnt = pl.pallas_call(
        _router_body,
        out_shape=(jax.ShapeDtypeStruct((N, 128), jnp.float32),
                   jax.ShapeDtypeStruct((1, 128), jnp.float32)),
        interpret=interpret,
    )(xf, wr_pad, br_pad)
    return duo, cnt


_SC_WIN = 128
_SC_SUB = 512


def _sc_dispatch(xw, tok, dest, R):
    A2 = dest.shape[1]
    W = xw.shape[1]
    xf2 = xw.reshape(-1, _SC_SUB)
    mesh = plsc.VectorSubcoreMesh(core_axis_name="core",
                                  subcore_axis_name="subcore")

    @functools.partial(
        pl.kernel,
        out_type=jax.ShapeDtypeStruct((R * (W // _SC_SUB), _SC_SUB),
                                      xw.dtype),
        mesh=mesh,
        scratch_types=[pltpu.VMEM((_SC_WIN, _SC_SUB), xw.dtype)])
    def k(x_hbm, tok_hbm, dest_hbm, xs_hbm, buf):
        def body(tok_vmem, dest_vmem):
            pltpu.sync_copy(x_hbm.at[tok_vmem.at[0]], buf)
            pltpu.sync_copy(buf, xs_hbm.at[dest_vmem.at[0]])

        pltpu.emit_pipeline(
            body,
            grid=(A2 // _SC_WIN,),
            in_specs=[pl.BlockSpec((1, _SC_WIN), lambda i: (0, i)),
                      pl.BlockSpec((1, _SC_WIN), lambda i: (0, i))],
            out_specs=[],
            core_axis_name=('core', 'subcore'),
            dimension_semantics=(pltpu.PARALLEL,),
        )(tok_hbm, dest_hbm)

    return k(xf2, tok, dest).reshape(R, W)


def kernel(x, Wr, br, W1, b1, W2, b2, interpret=False):
    B, L, D = x.shape
    xf = x.reshape(-1, D)
    N = xf.shape[0]
    A = N * TOP_K

    n_blocks = (A + N_EXPERTS * (BLK - 1) + BLK - 1) // BLK
    R = n_blocks * BLK

    duo, cnt = _router(xf, Wr, br, interpret=interpret)
    pos = duo[:, :TOP_K].astype(jnp.int32)
    tkw = duo[:, TOP_K:2 * TOP_K]
    dest = pos.reshape(-1)
    tok = jnp.arange(A, dtype=jnp.int32) // TOP_K

    pc = jnp.ceil(cnt[0, :N_EXPERTS] * (1.0 / BLK)) * BLK
    cum_pc = jnp.cumsum(pc)
    n_real = (cum_pc[-1] * (1.0 / BLK)).astype(jnp.int32)
    gid = jnp.minimum(
        jnp.searchsorted(
            cum_pc, (jnp.arange(n_blocks) * BLK).astype(jnp.float32),
            side='right'),
        N_EXPERTS - 1).astype(jnp.int32)
    gid_last = jnp.take(gid, jnp.maximum(n_real - 1, 0))
    gid = jnp.where(jnp.arange(n_blocks) < n_real, gid, gid_last)
    gid = jnp.concatenate([gid, n_real[None]])

    if interpret:
        xs = jnp.zeros((R, D), x.dtype).at[dest].set(xf[tok])
    else:
        nsub = D // _SC_SUB
        sub = jnp.arange(nsub, dtype=jnp.int32)
        tok2 = (tok[:, None] * nsub + sub).reshape(1, A * nsub)
        dest2 = (dest[:, None] * nsub + sub).reshape(1, A * nsub)
        xs = _sc_dispatch(xf, tok2, dest2, R)

    bf = jnp.bfloat16
    ys = _grouped_ffn(gid, xs, W1.astype(bf),
                      b1.reshape(N_EXPERTS, 1, D_FF), W2.astype(bf),
                      b2.reshape(N_EXPERTS, 1, D_MODEL), n_blocks,
                      interpret=interpret)

    out = ys[pos[:, 0]] * tkw[:, :1] + ys[pos[:, 1]] * tkw[:, 1:]
    return out.reshape(B, L, D)


# speedup vs baseline: 1.5948x; 1.1357x over previous
import functools
import math

import jax
import jax.numpy as jnp
from jax.experimental import pallas as pl
from jax.experimental.pallas import tpu as pltpu
from jax.experimental.pallas import tpu_sc as plsc

D_MODEL = 1024
N_EXPERTS = 8
TOP_K = 2
D_FF = 4 * D_MODEL

BLK = 256
_SQRT_HALF = 1.0 / math.sqrt(2.0)


def _ffn_body(gid_ref, xs_ref, w1_ref, b1_ref, w2_ref, b2_ref, out_ref):
    n_blocks = gid_ref.shape[0] - 1

    @pl.when(pl.program_id(0) < gid_ref[n_blocks])
    def _():
        xs = xs_ref[...].astype(w1_ref.dtype)
        h = jnp.dot(xs, w1_ref[0], preferred_element_type=jnp.float32)
        h = h + b1_ref[0, 0].astype(jnp.float32)
        h = 0.5 * h * (1.0 + jax.lax.erf(h * _SQRT_HALF))
        y = jnp.dot(h.astype(w1_ref.dtype),
                    w2_ref[0].astype(w1_ref.dtype),
                    preferred_element_type=jnp.float32)
        out_ref[...] = y + b2_ref[0, 0].astype(jnp.float32)


def _grouped_ffn(gid, xs, W1, b1, W2, b2, n_blocks, interpret=False):
    grid_spec = pltpu.PrefetchScalarGridSpec(
        num_scalar_prefetch=1,
        grid=(n_blocks,),
        in_specs=[
            pl.BlockSpec((BLK, D_MODEL), lambda i, gid: (i, 0)),
            pl.BlockSpec((1, D_MODEL, D_FF), lambda i, gid: (gid[i], 0, 0)),
            pl.BlockSpec((1, 1, D_FF), lambda i, gid: (gid[i], 0, 0)),
            pl.BlockSpec((1, D_FF, D_MODEL), lambda i, gid: (gid[i], 0, 0)),
            pl.BlockSpec((1, 1, D_MODEL), lambda i, gid: (gid[i], 0, 0)),
        ],
        out_specs=pl.BlockSpec((BLK, D_MODEL), lambda i, gid: (i, 0)),
    )
    return pl.pallas_call(
        _ffn_body,
        grid_spec=grid_spec,
        out_shape=jax.ShapeDtypeStruct((n_blocks * BLK, D_MODEL), jnp.float32),
        compiler_params=pltpu.CompilerParams(
            dimension_semantics=("arbitrary",)),
        interpret=interpret,
    )(gid, xs, W1, b1, W2, b2)


def _router_body(x_ref, wr_ref, br_ref, duo_ref, cnt_ref):
    logits = jnp.dot(x_ref[...], wr_ref[...],
                     preferred_element_type=jnp.float32) + br_ref[...]
    m = jnp.max(logits, axis=-1, keepdims=True)
    ex = jnp.exp(logits - m)
    sm = ex / jnp.sum(ex, axis=-1, keepdims=True)

    lane = jax.lax.broadcasted_iota(jnp.int32, sm.shape, 1)
    i1 = jnp.argmax(sm, axis=-1, keepdims=True)
    v1 = jnp.max(sm, axis=-1, keepdims=True)
    oh1 = (lane == i1)
    sm2 = jnp.where(oh1, -1.0, sm)
    i2 = jnp.argmax(sm2, axis=-1, keepdims=True)
    v2 = jnp.max(sm2, axis=-1, keepdims=True)
    oh2 = (lane == i2)
    s = v1 + v2
    w1, w2 = v1 / s, v2 / s

    ohs = oh1.astype(jnp.float32) + oh2.astype(jnp.float32)
    c = ohs
    k = 1
    while k < c.shape[0]:
        c = c + jnp.concatenate(
            [jnp.zeros((k, c.shape[1]), c.dtype), c[:-k]], axis=0)
        k *= 2
    excl = c - ohs
    cnt = jnp.sum(ohs, axis=0, keepdims=True)
    cnt_ref[...] = cnt

    pc = jnp.ceil(cnt * (1.0 / BLK)) * BLK
    cum = pc
    for kk in (1, 2, 4):
        cum = cum + jnp.concatenate(
            [jnp.zeros((1, kk), cum.dtype), cum[:, :-kk]], axis=1)
    pad_off = cum - pc

    rank1 = jnp.sum(excl * oh1, axis=-1, keepdims=True)
    rank2 = jnp.sum(excl * oh2, axis=-1, keepdims=True)
    off1 = jnp.sum(pad_off * oh1, axis=-1, keepdims=True)
    off2 = jnp.sum(pad_off * oh2, axis=-1, keepdims=True)
    d1 = off1 + rank1
    d2 = off2 + rank2

    duo_ref[...] = (jnp.where(lane == 0, d1, 0.0)
                    + jnp.where(lane == 1, d2, 0.0)
                    + jnp.where(lane == 2, w1, 0.0)
                    + jnp.where(lane == 3, w2, 0.0))


def _router(xf, Wr, br, interpret=False):
    N = xf.shape[0]
    wr_pad = jnp.zeros((D_MODEL, 128), Wr.dtype).at[:, :N_EXPERTS].set(Wr)
    br_pad = jnp.full((1, 128), -1e30, br.dtype).at[0, :N_EXPERTS].set(br)
    duo, cnt = pl.pallas_call(
        _router_body,
        out_shape=(jax.ShapeDtypeStruct((N, 128), jnp.float32),
                   jax.ShapeDtypeStruct((1, 128), jnp.float32)),
        interpret=interpret,
    )(xf, wr_pad, br_pad)
    return duo, cnt


_SC_WIN = 128
_SC_SUB = 512


def _sc_dispatch(xw, tok, dest, R):
    A2 = dest.shape[1]
    W = xw.shape[1]
    xf2 = xw.reshape(-1, _SC_SUB)
    mesh = plsc.VectorSubcoreMesh(core_axis_name="core",
                                  subcore_axis_name="subcore")

    @functools.partial(
        pl.kernel,
        out_type=jax.ShapeDtypeStruct((R * (W // _SC_SUB), _SC_SUB),
                                      xw.dtype),
        mesh=mesh,
        scratch_types=[pltpu.VMEM((_SC_WIN, _SC_SUB), xw.dtype)])
    def k(x_hbm, tok_hbm, dest_hbm, xs_hbm, buf):
        def body(tok_vmem, dest_vmem):
            pltpu.sync_copy(x_hbm.at[tok_vmem.at[0]], buf)
            pltpu.sync_copy(buf, xs_hbm.at[dest_vmem.at[0]])

        pltpu.emit_pipeline(
            body,
            grid=(A2 // _SC_WIN,),
            in_specs=[pl.BlockSpec((1, _SC_WIN), lambda i: (0, i)),
                      pl.BlockSpec((1, _SC_WIN), lambda i: (0, i))],
            out_specs=[],
            core_axis_name=('core', 'subcore'),
            dimension_semantics=(pltpu.PARALLEL,),
        )(tok_hbm, dest_hbm)

    return k(xf2, tok, dest).reshape(R, W)


def kernel(x, Wr, br, W1, b1, W2, b2, interpret=False):
    B, L, D = x.shape
    xf = x.reshape(-1, D)
    N = xf.shape[0]
    A = N * TOP_K

    n_blocks = (A + N_EXPERTS * (BLK - 1) + BLK - 1) // BLK
    R = n_blocks * BLK

    duo, cnt = _router(xf, Wr, br, interpret=interpret)
    pos = duo[:, :TOP_K].astype(jnp.int32)
    tkw = duo[:, TOP_K:2 * TOP_K]
    dest = pos.reshape(-1)
    tok = jnp.arange(A, dtype=jnp.int32) // TOP_K

    pc = jnp.ceil(cnt[0, :N_EXPERTS] * (1.0 / BLK)) * BLK
    cum_pc = jnp.cumsum(pc)
    n_real = (cum_pc[-1] * (1.0 / BLK)).astype(jnp.int32)
    gid = jnp.minimum(
        jnp.searchsorted(
            cum_pc, (jnp.arange(n_blocks) * BLK).astype(jnp.float32),
            side='right'),
        N_EXPERTS - 1).astype(jnp.int32)
    gid_last = jnp.take(gid, jnp.maximum(n_real - 1, 0))
    gid = jnp.where(jnp.arange(n_blocks) < n_real, gid, gid_last)
    gid = jnp.concatenate([gid, n_real[None]])

    if interpret:
        xs = jnp.zeros((R, D), x.dtype).at[dest].set(xf[tok])
    else:
        nsub = D // _SC_SUB
        sub = jnp.arange(nsub, dtype=jnp.int32)
        tok2 = (tok[:, None] * nsub + sub).reshape(1, A * nsub)
        dest2 = (dest[:, None] * nsub + sub).reshape(1, A * nsub)
        xs = _sc_dispatch(xf, tok2, dest2, R)

    bf = jnp.bfloat16
    ys = _grouped_ffn(gid, xs, W1.astype(bf),
                      b1.reshape(N_EXPERTS, 1, D_FF), W2,
                      b2.reshape(N_EXPERTS, 1, D_MODEL), n_blocks,
                      interpret=interpret)

    out = ys[pos[:, 0]] * tkw[:, :1] + ys[pos[:, 1]] * tkw[:, 1:]
    return out.reshape(B, L, D)
